# baseline (speedup 1.0000x reference)
import jax
import jax.numpy as jnp
from jax import lax
from jax.experimental import pallas as pl
from jax.experimental.pallas import tpu as pltpu
from jax.experimental.pallas import tpu_sc as plsc

_B, _T = 8, 2048
_L = 16
_NW = 2 * _B
_CHUNK = _T // 2
_NBLK = _CHUNK // _L + 1
_MIN_GAP = 5
_MIN_SIZE = 5
_NEG = float("-inf")


def _bcast_lane(v, i):
    return v.at[jnp.full((_L,), i, jnp.int32)].get(mode="promise_in_bounds")


def _gather_lanes(v, idx):
    return v.at[idx].get(mode="promise_in_bounds")


def _loss_body(w_hbm, ypk_hbm, stage_hbm, out_hbm, wv, yv, resv, acc16):
    c = lax.axis_index("c")
    s = lax.axis_index("s")
    lane = lax.iota(jnp.int32, _L)

    rr = s // 2
    kk = s - 2 * rr
    start = 1008 * kk

    @pl.when(c == 0)
    def _compute():
        pltpu.sync_copy(w_hbm.at[rr], wv)
        pltpu.sync_copy(ypk_hbm.at[rr], yv)
        yvec = yv[...]
        y0 = _bcast_lane(yvec, 0)
        y1 = _bcast_lane(yvec, 1)
        ev = _bcast_lane(yvec, 2)
        fm_cap = 1024 * kk + _CHUNK + _L - _MIN_SIZE

        def pen(iv, yc):
            t = jnp.maximum(jnp.abs(yc - iv) - ev, 0)
            return t.astype(jnp.float32) * jnp.float32(0.5)

        def body(t, carry):
            s_next, best, fm, e = carry
            bb = start + (_NBLK - 1 - t) * _L
            wvec = wv[pl.ds(bb, _L)]
            iv = bb + lane
            inb = iv < _T
            fv = wvec + pen(iv, y0)
            gv = jnp.where(inb, wvec + pen(iv, y1), _NEG)
            wsuf = jnp.flip(plsc.cummax(jnp.flip(gv, 0)), 0)
            s_cur = jnp.maximum(wsuf, _bcast_lane(s_next, 0))
            h_lo = _gather_lanes(s_cur, jnp.minimum(lane + _MIN_SIZE, _L - 1))
            h_hi = _gather_lanes(s_next, jnp.maximum(lane - (_L - _MIN_SIZE), 0))
            h = jnp.where(lane < _L - _MIN_SIZE, h_lo, h_hi)
            r = jnp.where((iv >= _MIN_GAP) & inb, fv + h, _NEG)
            best = jnp.maximum(best, r)
            fm = jnp.maximum(
                fm, jnp.where((iv >= _MIN_GAP) & (iv <= fm_cap), fv, _NEG))
            e = jnp.where(t == _NBLK - 2 - kk, s_cur, e)
            return s_cur, best, fm, e

        neg = jnp.full((_L,), _NEG, jnp.float32)
        _, best, fm, e = lax.fori_loop(0, _NBLK, body, (neg, neg, neg, neg))

        w0 = wv[pl.ds(0, _L)]
        f0 = w0 + pen(lane, y0)
        g0 = w0 + pen(lane, y1)
        init = _bcast_lane(f0, 1) + _bcast_lane(g0, 1 + _MIN_SIZE)
        best = jnp.where(kk == 0, jnp.maximum(best, init), best)

        bs = jnp.broadcast_to(jnp.max(best), (_L,))
        fs = jnp.broadcast_to(jnp.max(fm), (_L,))
        es = _bcast_lane(e, 0)
        resv[...] = jnp.where(lane < 5, bs, jnp.where(lane < 10, fs, es))
        pltpu.sync_copy(resv, stage_hbm.at[s])

    plsc.subcore_barrier()

    @pl.when(jnp.logical_and(c == 0, s == 0))
    def _reduce():
        pltpu.sync_copy(stage_hbm, acc16)
        acc = jnp.zeros((_L,), jnp.float32)
        for r in range(_B):
            row0 = acc16[2 * r]
            row1 = acc16[2 * r + 1]
            b0 = _bcast_lane(row0, 0)
            f0 = _bcast_lane(row0, 5)
            b1 = _bcast_lane(row1, 0)
            e1 = _bcast_lane(row1, 10)
            acc = acc + jnp.maximum(jnp.maximum(b0, b1), f0 + e1)
        resv[...] = acc * jnp.float32(1.0 / _B)
        pltpu.sync_copy(resv, out_hbm)


_sc_loss = pl.kernel(
    _loss_body,
    out_type=(
        jax.ShapeDtypeStruct((_NW, _L), jnp.float32),
        jax.ShapeDtypeStruct((_L,), jnp.float32),
    ),
    mesh=plsc.VectorSubcoreMesh(core_axis_name="c", subcore_axis_name="s",
                                num_cores=1, num_subcores=16),
    scratch_types=[
        pltpu.VMEM((_T,), jnp.float32),
        pltpu.VMEM((_L,), jnp.int32),
        pltpu.VMEM((_L,), jnp.float32),
        pltpu.VMEM((_NW, _L), jnp.float32),
    ],
    compiler_params=pltpu.CompilerParams(needs_layout_passes=False),
)


def kernel(w_phi, y, eps):
    ypk = jnp.pad(y.astype(jnp.int32), ((0, 0), (0, _L - 2)),
                  constant_values=jnp.asarray(eps, jnp.int32))
    _, out = _sc_loss(w_phi, ypk)
    return out[0]


# speedup vs baseline: 1.0255x; 1.0255x over previous
import jax
import jax.numpy as jnp
from jax import lax
from jax.experimental import pallas as pl
from jax.experimental.pallas import tpu as pltpu
from jax.experimental.pallas import tpu_sc as plsc

_B, _T = 8, 2048
_L = 16
_NW = 2 * _B
_CHUNK = _T // 2
_NBLK = _CHUNK // _L + 1
_MIN_GAP = 5
_MIN_SIZE = 5
_NEG = float("-inf")


def _bcast_lane(v, i):
    return v.at[jnp.full((_L,), i, jnp.int32)].get(mode="promise_in_bounds")


def _gather_lanes(v, idx):
    return v.at[idx].get(mode="promise_in_bounds")


def _loss_body(w_hbm, ypk_hbm, stage_hbm, out_hbm, wv, yv, resv, acc16,
               sem_w, sem_y):
    c = lax.axis_index("c")
    s = lax.axis_index("s")
    lane = lax.iota(jnp.int32, _L)

    rr = s // 2
    kk = s - 2 * rr
    start = 1008 * kk

    @pl.when(c == 0)
    def _compute():
        cw = pltpu.async_copy(w_hbm.at[rr], wv, sem_w)
        cy = pltpu.async_copy(ypk_hbm.at[rr], yv, sem_y)
        cw.wait()
        cy.wait()
        yvec = yv[...]
        y0 = _bcast_lane(yvec, 0)
        y1 = _bcast_lane(yvec, 1)
        ev = _bcast_lane(yvec, 2)
        fm_cap = 1024 * kk + _CHUNK + _L - _MIN_SIZE

        def pen(iv, yc):
            t = jnp.maximum(jnp.abs(yc - iv) - ev, 0)
            return t.astype(jnp.float32) * jnp.float32(0.5)

        def body(t, carry):
            s_next, best, fm, e = carry
            bb = start + (_NBLK - 1 - t) * _L
            wvec = wv[pl.ds(bb, _L)]
            iv = bb + lane
            inb = iv < _T
            fv = wvec + pen(iv, y0)
            gv = jnp.where(inb, wvec + pen(iv, y1), _NEG)
            wsuf = jnp.flip(plsc.cummax(jnp.flip(gv, 0)), 0)
            s_cur = jnp.maximum(wsuf, _bcast_lane(s_next, 0))
            h_lo = _gather_lanes(s_cur, jnp.minimum(lane + _MIN_SIZE, _L - 1))
            h_hi = _gather_lanes(s_next, jnp.maximum(lane - (_L - _MIN_SIZE), 0))
            h = jnp.where(lane < _L - _MIN_SIZE, h_lo, h_hi)
            r = jnp.where((iv >= _MIN_GAP) & inb, fv + h, _NEG)
            best = jnp.maximum(best, r)
            fm = jnp.maximum(
                fm, jnp.where((iv >= _MIN_GAP) & (iv <= fm_cap), fv, _NEG))
            e = jnp.where(t == _NBLK - 2 - kk, s_cur, e)
            return s_cur, best, fm, e

        neg = jnp.full((_L,), _NEG, jnp.float32)
        _, best, fm, e = lax.fori_loop(0, _NBLK, body, (neg, neg, neg, neg))

        w0 = wv[pl.ds(0, _L)]
        f0 = w0 + pen(lane, y0)
        g0 = w0 + pen(lane, y1)
        init = _bcast_lane(f0, 1) + _bcast_lane(g0, 1 + _MIN_SIZE)
        best = jnp.where(kk == 0, jnp.maximum(best, init), best)

        bs = jnp.broadcast_to(jnp.max(best), (_L,))
        fs = jnp.broadcast_to(jnp.max(fm), (_L,))
        es = _bcast_lane(e, 0)
        resv[...] = jnp.where(lane < 5, bs, jnp.where(lane < 10, fs, es))
        pltpu.sync_copy(resv, stage_hbm.at[s])

    plsc.subcore_barrier()

    @pl.when(jnp.logical_and(c == 0, s == 0))
    def _reduce():
        pltpu.sync_copy(stage_hbm, acc16)
        acc = jnp.zeros((_L,), jnp.float32)
        for r in range(_B):
            row0 = acc16[2 * r]
            row1 = acc16[2 * r + 1]
            b0 = _bcast_lane(row0, 0)
            f0 = _bcast_lane(row0, 5)
            b1 = _bcast_lane(row1, 0)
            e1 = _bcast_lane(row1, 10)
            acc = acc + jnp.maximum(jnp.maximum(b0, b1), f0 + e1)
        resv[...] = acc * jnp.float32(1.0 / _B)
        pltpu.sync_copy(resv, out_hbm)


_sc_loss = pl.kernel(
    _loss_body,
    out_type=(
        jax.ShapeDtypeStruct((_NW, _L), jnp.float32),
        jax.ShapeDtypeStruct((_L,), jnp.float32),
    ),
    mesh=plsc.VectorSubcoreMesh(core_axis_name="c", subcore_axis_name="s",
                                num_cores=1, num_subcores=16),
    scratch_types=[
        pltpu.VMEM((_T,), jnp.float32),
        pltpu.VMEM((_L,), jnp.int32),
        pltpu.VMEM((_L,), jnp.float32),
        pltpu.VMEM((_NW, _L), jnp.float32),
        pltpu.SemaphoreType.DMA,
        pltpu.SemaphoreType.DMA,
    ],
    compiler_params=pltpu.CompilerParams(needs_layout_passes=False),
)


def kernel(w_phi, y, eps):
    ypk = jnp.pad(y.astype(jnp.int32), ((0, 0), (0, _L - 2)),
                  constant_values=jnp.asarray(eps, jnp.int32))
    _, out = _sc_loss(w_phi, ypk)
    return out[0]
